# baseline (speedup 1.0000x reference)
import functools

import jax
import jax.numpy as jnp
from jax.experimental import pallas as pl
from jax.experimental.pallas import tpu as pltpu

DIM_ = 96
NHEADS_ = 4
NWIN_ = 28
HD_ = DIM_ // NHEADS_
NREG_ = NWIN_ * NWIN_
RSS_ = 64
TOPK_ = 2
K2_ = TOPK_ * RSS_ // 8
NEG_ = -1e30

B_ = 2
HW_ = 224


def _qkv_kernel(x_ref, w_ref, b_ref, qT_ref, kT_ref, vT_ref, vx_ref,
                pool_ref, *, regs):
    x = x_ref[0].reshape(8 * HW_, DIM_)
    acc = jnp.dot(x, w_ref[...], preferred_element_type=jnp.float32)
    acc = acc + b_ref[...]
    acc4 = acc.reshape(8, regs, 8, 3 * DIM_)
    pool_ref[0] = jnp.mean(acc4, axis=(0, 2))[:, :2 * DIM_]
    accr = acc4.transpose(1, 0, 2, 3).reshape(regs, RSS_, 3 * DIM_)
    accT = jnp.swapaxes(accr, 1, 2)
    for h in range(NHEADS_):
        qT_ref[0, h] = accT[:, h * HD_:(h + 1) * HD_, :]
        kT_ref[0, h] = accT[:, DIM_ + h * HD_:DIM_ + (h + 1) * HD_, :]
        vT_ref[0, h] = accT[:, 2 * DIM_ + h * HD_:2 * DIM_ + (h + 1) * HD_, :]
    v3 = acc[:, 2 * DIM_:].reshape(8, HW_, DIM_)
    z = jnp.zeros((8, 1, DIM_), jnp.float32)
    vx_ref[0] = jnp.concatenate([z, v3, z], axis=1)


def _qkv_proj(x, wT, b2d):
    regs = NWIN_
    nchunk = NREG_ // regs
    grid = B_ * nchunk
    tblspec = pl.BlockSpec((1, NHEADS_, regs, HD_, RSS_),
                           lambda t: (t // nchunk, 0, t % nchunk, 0, 0))
    tblshape = jax.ShapeDtypeStruct((B_, NHEADS_, NREG_, HD_, RSS_), jnp.float32)
    return pl.pallas_call(
        functools.partial(_qkv_kernel, regs=regs),
        grid=(grid,),
        in_specs=[
            pl.BlockSpec((1, 8, HW_, DIM_),
                         lambda t: (t // nchunk, t % nchunk, 0, 0)),
            pl.BlockSpec((DIM_, 3 * DIM_), lambda t: (0, 0)),
            pl.BlockSpec((1, 3 * DIM_), lambda t: (0, 0)),
        ],
        out_specs=[
            tblspec, tblspec, tblspec,
            pl.BlockSpec((1, 8, HW_ + 2, DIM_),
                         lambda t: (t // nchunk, t % nchunk, 0, 0)),
            pl.BlockSpec((1, regs, 2 * DIM_), lambda t: (t, 0, 0)),
        ],
        out_shape=[
            tblshape, tblshape, tblshape,
            jax.ShapeDtypeStruct((B_, HW_, HW_ + 2, DIM_), jnp.float32),
            jax.ShapeDtypeStruct((grid, regs, 2 * DIM_), jnp.float32),
        ],
    )(x, wT, b2d)


def _route_kernel(qk_ref, kr_ref, i0_ref, i1_ref):
    q = qk_ref[:, :DIM_]
    kr = kr_ref[0, :, DIM_:]
    a = jax.lax.dot_general(q, kr, (((1,), (1,)), ((), ())),
                            preferred_element_type=jnp.float32)
    iota = jax.lax.broadcasted_iota(jnp.int32, a.shape, 1)
    m0 = jnp.max(a, axis=1, keepdims=True)
    i0 = jnp.min(jnp.where(a >= m0, iota, NREG_), axis=1, keepdims=True)
    a2 = jnp.where(iota == i0, NEG_, a)
    m1 = jnp.max(a2, axis=1, keepdims=True)
    i1 = jnp.min(jnp.where(a2 >= m1, iota, NREG_), axis=1, keepdims=True)
    i0_ref[0] = i0
    i1_ref[0] = i1


def _route(qk_flat, pooled3):
    rows = 112
    nchunk = NREG_ // rows
    grid = B_ * nchunk
    i0, i1 = pl.pallas_call(
        _route_kernel,
        grid=(grid,),
        in_specs=[
            pl.BlockSpec((rows, 2 * DIM_), lambda t: (t, 0)),
            pl.BlockSpec((1, NREG_, 2 * DIM_), lambda t: (t // nchunk, 0, 0)),
        ],
        out_specs=[
            pl.BlockSpec((1, rows, 1), lambda t: (t, 0, 0)),
            pl.BlockSpec((1, rows, 1), lambda t: (t, 0, 0)),
        ],
        out_shape=[
            jax.ShapeDtypeStruct((grid, rows, 1), jnp.int32),
            jax.ShapeDtypeStruct((grid, rows, 1), jnp.int32),
        ],
    )(qk_flat, pooled3)
    return i0.reshape(B_, NREG_), i1.reshape(B_, NREG_)


def _attn_kernel(i0_ref, i1_ref, q_ref, k_ref, v_ref, o_ref, *, scale):
    bh = pl.program_id(0)
    rh = pl.program_id(1)
    b = bh // NHEADS_
    npair = NWIN_ // 2

    dn = (((0,), (0,)), ((), ()))
    dn2 = (((0,), (1,)), ((), ()))
    gsize = 7
    outs = [None] * NWIN_
    for g in range(npair // gsize):
        idx = []
        origs = []
        for p in range(g * gsize, (g + 1) * gsize):
            rA = rh * NWIN_ + 2 * p
            rB = rA + 1
            qA = q_ref[0, 2 * p] * scale
            qB = q_ref[0, 2 * p + 1] * scale
            iA0 = i0_ref[b, rA]
            iA1 = i1_ref[b, rA]
            iB0 = i0_ref[b, rB]
            iB1 = i1_ref[b, rB]
            idx.append((iA0, iA1, iB0, iB1))
            tA = jnp.concatenate(
                [jax.lax.dot_general(k_ref[0, iA0], qA, dn, preferred_element_type=jnp.float32),
                 jax.lax.dot_general(k_ref[0, iA1], qA, dn, preferred_element_type=jnp.float32)],
                axis=0)
            tB = jnp.concatenate(
                [jax.lax.dot_general(k_ref[0, iB0], qB, dn, preferred_element_type=jnp.float32),
                 jax.lax.dot_general(k_ref[0, iB1], qB, dn, preferred_element_type=jnp.float32)],
                axis=0)
            origs.append(jnp.concatenate([tA, tB], axis=1))

        ts = list(origs)
        ms = [None] * gsize
        for it in range(K2_ - 1):
            for j in range(gsize):
                cm = jnp.max(ts[j], axis=0, keepdims=True)
                if it == 0:
                    ms[j] = cm
                ts[j] = jnp.where(ts[j] >= cm, NEG_, ts[j])

        for j in range(gsize):
            p = g * gsize + j
            orig = origs[j]
            iA0, iA1, iB0, iB1 = idx[j]
            thresh = jnp.max(ts[j], axis=0, keepdims=True)
            e = jnp.where(orig >= thresh, jnp.exp(orig - ms[j]), 0.0)
            s = jnp.sum(e, axis=0, keepdims=True)
            pn = e / s
            vA = jnp.concatenate([v_ref[0, iA0], v_ref[0, iA1]], axis=1)
            vB = jnp.concatenate([v_ref[0, iB0], v_ref[0, iB1]], axis=1)
            oA = jax.lax.dot_general(pn[:, :64], vA, dn2, preferred_element_type=jnp.float32)
            oB = jax.lax.dot_general(pn[:, 64:], vB, dn2, preferred_element_type=jnp.float32)
            outs[2 * p] = oA
            outs[2 * p + 1] = oB
    o4 = jnp.stack(outs).reshape(NWIN_, 8, 8, HD_).transpose(1, 0, 2, 3)
    o_ref[0, 0] = o4.reshape(8, HW_, HD_)


def _core_attn(qT, kT, vT, i0, i1):
    scale = DIM_ ** (-0.5)
    grid_spec = pltpu.PrefetchScalarGridSpec(
        num_scalar_prefetch=2,
        grid=(B_ * NHEADS_, NWIN_),
        in_specs=[
            pl.BlockSpec((1, NWIN_, HD_, RSS_), lambda bh, c, i0, i1: (bh, c, 0, 0)),
            pl.BlockSpec((1, NREG_, HD_, RSS_), lambda bh, c, i0, i1: (bh, 0, 0, 0)),
            pl.BlockSpec((1, NREG_, HD_, RSS_), lambda bh, c, i0, i1: (bh, 0, 0, 0)),
        ],
        out_specs=pl.BlockSpec(
            (1, 1, 8, HW_, HD_),
            lambda bh, c, i0, i1: (bh // NHEADS_, bh % NHEADS_, c, 0, 0)),
    )
    return pl.pallas_call(
        functools.partial(_attn_kernel, scale=scale),
        grid_spec=grid_spec,
        out_shape=jax.ShapeDtypeStruct((B_, NHEADS_, HW_, HW_, HD_), jnp.float32),
    )(i0, i1, qT, kT, vT)


def _lepe_kernel(vp_ref, vc_ref, vn_ref, a0_ref, a1_ref, a2_ref, a3_ref,
                 wl_ref, bl_ref, woT_ref, bo_ref, o_ref, *, th, ntile):
    t = pl.program_id(1)
    acc = jnp.concatenate(
        [a0_ref[0, 0], a1_ref[0, 0], a2_ref[0, 0], a3_ref[0, 0]], axis=-1)
    cur = vc_ref[0]
    prev_last = vp_ref[0, th - 1:th] * jnp.where(t > 0, 1.0, 0.0)
    next_first = vn_ref[0, 0:1] * jnp.where(t < ntile - 1, 1.0, 0.0)
    rows = [
        jnp.concatenate([prev_last, cur[:th - 1]], axis=0),
        cur,
        jnp.concatenate([cur[1:], next_first], axis=0),
    ]
    for dy in range(3):
        vd = rows[dy]
        for dx in range(3):
            win = vd[:, dx:dx + HW_, :]
            lw = wl_ref[dy * 3 + dx][None, None, :]
            acc = acc + win * lw
    acc = acc + bl_ref[...][None]
    dn = (((2,), (0,)), ((), ()))
    out = jax.lax.dot_general(acc, woT_ref[...], dn, preferred_element_type=jnp.float32)
    o_ref[0] = out + bo_ref[...][None]


def _lepe_proj(vx, attn4, wl, bl2d, woT, bo2d):
    th = 28
    ntile = HW_ // th
    grid = (B_, ntile)
    vspec_p = pl.BlockSpec((1, th, HW_ + 2, DIM_),
                           lambda b, t: (b, jnp.maximum(t - 1, 0), 0, 0))
    vspec_c = pl.BlockSpec((1, th, HW_ + 2, DIM_), lambda b, t: (b, t, 0, 0))
    vspec_n = pl.BlockSpec((1, th, HW_ + 2, DIM_),
                           lambda b, t: (b, jnp.minimum(t + 1, ntile - 1), 0, 0))
    aspec = [pl.BlockSpec((1, 1, th, HW_, HD_),
                          lambda b, t, h=h: (b, h, t, 0, 0)) for h in range(NHEADS_)]
    return pl.pallas_call(
        functools.partial(_lepe_kernel, th=th, ntile=ntile),
        grid=grid,
        in_specs=[
            vspec_p, vspec_c, vspec_n,
            *aspec,
            pl.BlockSpec((9, DIM_), lambda b, t: (0, 0)),
            pl.BlockSpec((1, DIM_), lambda b, t: (0, 0)),
            pl.BlockSpec((DIM_, DIM_), lambda b, t: (0, 0)),
            pl.BlockSpec((1, DIM_), lambda b, t: (0, 0)),
        ],
        out_specs=pl.BlockSpec((1, th, HW_, DIM_), lambda b, t: (b, t, 0, 0)),
        out_shape=jax.ShapeDtypeStruct((B_, HW_, HW_, DIM_), jnp.float32),
    )(vx, vx, vx, attn4, attn4, attn4, attn4, wl, bl2d, woT, bo2d)


def kernel(x, Wqkv, bqkv, Wlepe, blepe, Wout, bout):
    qT5, kT5, vT5, vx, pooled = _qkv_proj(
        x, Wqkv.T, bqkv.reshape(1, 3 * DIM_))
    qk_flat = pooled.reshape(B_ * NREG_, 2 * DIM_)
    pooled3 = pooled.reshape(B_, NREG_, 2 * DIM_)

    i0, i1 = _route(qk_flat, pooled3)

    qT = qT5.reshape(B_ * NHEADS_, NREG_, HD_, RSS_)
    kT = kT5.reshape(B_ * NHEADS_, NREG_, HD_, RSS_)
    vT = vT5.reshape(B_ * NHEADS_, NREG_, HD_, RSS_)

    attn4 = _core_attn(qT, kT, vT, i0, i1)

    wl = Wlepe.reshape(DIM_, 9).T
    out = _lepe_proj(vx, attn4, wl, blepe.reshape(1, DIM_),
                     Wout.T, bout.reshape(1, DIM_))
    return out


# speedup vs baseline: 1.0356x; 1.0356x over previous
import functools

import jax
import jax.numpy as jnp
from jax.experimental import pallas as pl
from jax.experimental.pallas import tpu as pltpu

DIM_ = 96
NHEADS_ = 4
NWIN_ = 28
HD_ = DIM_ // NHEADS_
NREG_ = NWIN_ * NWIN_
RSS_ = 64
TOPK_ = 2
K2_ = TOPK_ * RSS_ // 8
NEG_ = -1e30

B_ = 2
HW_ = 224


def _qkv_kernel(x_ref, w_ref, b_ref, qT_ref, kT_ref, vT_ref, vx_ref,
                pool_ref, *, regs):
    x = x_ref[0].reshape(8 * HW_, DIM_)
    acc = jnp.dot(x, w_ref[...], preferred_element_type=jnp.float32)
    acc = acc + b_ref[...]
    acc4 = acc.reshape(8, regs, 8, 3 * DIM_)
    pool_ref[0] = jnp.mean(acc4, axis=(0, 2))[:, :2 * DIM_]
    accr = acc4.transpose(1, 0, 2, 3).reshape(regs, RSS_, 3 * DIM_)
    accT = jnp.swapaxes(accr, 1, 2)
    for h in range(NHEADS_):
        qT_ref[0, h] = accT[:, h * HD_:(h + 1) * HD_, :]
        kT_ref[0, h] = accT[:, DIM_ + h * HD_:DIM_ + (h + 1) * HD_, :]
        vT_ref[0, h] = accT[:, 2 * DIM_ + h * HD_:2 * DIM_ + (h + 1) * HD_, :]
    v3 = acc[:, 2 * DIM_:].reshape(8, HW_, DIM_)
    z = jnp.zeros((8, 1, DIM_), jnp.float32)
    vx_ref[0] = jnp.concatenate([z, v3, z], axis=1)


def _qkv_proj(x, wT, b2d):
    regs = NWIN_
    nchunk = NREG_ // regs
    grid = B_ * nchunk
    tblspec = pl.BlockSpec((1, NHEADS_, regs, HD_, RSS_),
                           lambda t: (t // nchunk, 0, t % nchunk, 0, 0))
    tblshape = jax.ShapeDtypeStruct((B_, NHEADS_, NREG_, HD_, RSS_), jnp.float32)
    return pl.pallas_call(
        functools.partial(_qkv_kernel, regs=regs),
        grid=(grid,),
        in_specs=[
            pl.BlockSpec((1, 8, HW_, DIM_),
                         lambda t: (t // nchunk, t % nchunk, 0, 0)),
            pl.BlockSpec((DIM_, 3 * DIM_), lambda t: (0, 0)),
            pl.BlockSpec((1, 3 * DIM_), lambda t: (0, 0)),
        ],
        out_specs=[
            tblspec, tblspec, tblspec,
            pl.BlockSpec((1, 8, HW_ + 2, DIM_),
                         lambda t: (t // nchunk, t % nchunk, 0, 0)),
            pl.BlockSpec((1, regs, 2 * DIM_), lambda t: (t, 0, 0)),
        ],
        out_shape=[
            tblshape, tblshape, tblshape,
            jax.ShapeDtypeStruct((B_, HW_, HW_ + 2, DIM_), jnp.float32),
            jax.ShapeDtypeStruct((grid, regs, 2 * DIM_), jnp.float32),
        ],
    )(x, wT, b2d)


def _route_kernel(qk_ref, kr_ref, i0_ref, i1_ref):
    q = qk_ref[:, :DIM_]
    kr = kr_ref[0, :, DIM_:]
    a = jax.lax.dot_general(q, kr, (((1,), (1,)), ((), ())),
                            preferred_element_type=jnp.float32)
    iota = jax.lax.broadcasted_iota(jnp.int32, a.shape, 1)
    m0 = jnp.max(a, axis=1, keepdims=True)
    i0 = jnp.min(jnp.where(a >= m0, iota, NREG_), axis=1, keepdims=True)
    a2 = jnp.where(iota == i0, NEG_, a)
    m1 = jnp.max(a2, axis=1, keepdims=True)
    i1 = jnp.min(jnp.where(a2 >= m1, iota, NREG_), axis=1, keepdims=True)
    i0_ref[0] = i0
    i1_ref[0] = i1


def _route(qk_flat, pooled3):
    rows = 112
    nchunk = NREG_ // rows
    grid = B_ * nchunk
    i0, i1 = pl.pallas_call(
        _route_kernel,
        grid=(grid,),
        in_specs=[
            pl.BlockSpec((rows, 2 * DIM_), lambda t: (t, 0)),
            pl.BlockSpec((1, NREG_, 2 * DIM_), lambda t: (t // nchunk, 0, 0)),
        ],
        out_specs=[
            pl.BlockSpec((1, rows, 1), lambda t: (t, 0, 0)),
            pl.BlockSpec((1, rows, 1), lambda t: (t, 0, 0)),
        ],
        out_shape=[
            jax.ShapeDtypeStruct((grid, rows, 1), jnp.int32),
            jax.ShapeDtypeStruct((grid, rows, 1), jnp.int32),
        ],
    )(qk_flat, pooled3)
    return i0.reshape(B_, NREG_), i1.reshape(B_, NREG_)


def _attn_kernel(i0_ref, i1_ref, q_ref, k_ref, v_ref, o_ref, *, scale):
    bh = pl.program_id(0)
    rh = pl.program_id(1)
    b = bh // NHEADS_
    npair = NWIN_ // 2

    dn = (((0,), (0,)), ((), ()))
    dn2 = (((0,), (1,)), ((), ()))
    gsize = npair
    outs = [None] * NWIN_
    for g in range(npair // gsize):
        idx = []
        origs = []
        for p in range(g * gsize, (g + 1) * gsize):
            rA = rh * NWIN_ + 2 * p
            rB = rA + 1
            qA = q_ref[0, 2 * p] * scale
            qB = q_ref[0, 2 * p + 1] * scale
            iA0 = i0_ref[b, rA]
            iA1 = i1_ref[b, rA]
            iB0 = i0_ref[b, rB]
            iB1 = i1_ref[b, rB]
            idx.append((iA0, iA1, iB0, iB1))
            tA = jnp.concatenate(
                [jax.lax.dot_general(k_ref[0, iA0], qA, dn, preferred_element_type=jnp.float32),
                 jax.lax.dot_general(k_ref[0, iA1], qA, dn, preferred_element_type=jnp.float32)],
                axis=0)
            tB = jnp.concatenate(
                [jax.lax.dot_general(k_ref[0, iB0], qB, dn, preferred_element_type=jnp.float32),
                 jax.lax.dot_general(k_ref[0, iB1], qB, dn, preferred_element_type=jnp.float32)],
                axis=0)
            origs.append(jnp.concatenate([tA, tB], axis=1))

        ts = list(origs)
        ms = [None] * gsize
        for it in range(K2_ - 1):
            for j in range(gsize):
                cm = jnp.max(ts[j], axis=0, keepdims=True)
                if it == 0:
                    ms[j] = cm
                ts[j] = jnp.where(ts[j] >= cm, NEG_, ts[j])

        for j in range(gsize):
            p = g * gsize + j
            orig = origs[j]
            iA0, iA1, iB0, iB1 = idx[j]
            thresh = jnp.max(ts[j], axis=0, keepdims=True)
            e = jnp.where(orig >= thresh, jnp.exp(orig - ms[j]), 0.0)
            s = jnp.sum(e, axis=0, keepdims=True)
            pn = e / s
            vA = jnp.concatenate([v_ref[0, iA0], v_ref[0, iA1]], axis=1)
            vB = jnp.concatenate([v_ref[0, iB0], v_ref[0, iB1]], axis=1)
            oA = jax.lax.dot_general(pn[:, :64], vA, dn2, preferred_element_type=jnp.float32)
            oB = jax.lax.dot_general(pn[:, 64:], vB, dn2, preferred_element_type=jnp.float32)
            outs[2 * p] = oA
            outs[2 * p + 1] = oB
    o4 = jnp.stack(outs).reshape(NWIN_, 8, 8, HD_).transpose(1, 0, 2, 3)
    o_ref[0, 0] = o4.reshape(8, HW_, HD_)


def _core_attn(qT, kT, vT, i0, i1):
    scale = DIM_ ** (-0.5)
    grid_spec = pltpu.PrefetchScalarGridSpec(
        num_scalar_prefetch=2,
        grid=(B_ * NHEADS_, NWIN_),
        in_specs=[
            pl.BlockSpec((1, NWIN_, HD_, RSS_), lambda bh, c, i0, i1: (bh, c, 0, 0)),
            pl.BlockSpec((1, NREG_, HD_, RSS_), lambda bh, c, i0, i1: (bh, 0, 0, 0)),
            pl.BlockSpec((1, NREG_, HD_, RSS_), lambda bh, c, i0, i1: (bh, 0, 0, 0)),
        ],
        out_specs=pl.BlockSpec(
            (1, 1, 8, HW_, HD_),
            lambda bh, c, i0, i1: (bh // NHEADS_, bh % NHEADS_, c, 0, 0)),
    )
    return pl.pallas_call(
        functools.partial(_attn_kernel, scale=scale),
        grid_spec=grid_spec,
        out_shape=jax.ShapeDtypeStruct((B_, NHEADS_, HW_, HW_, HD_), jnp.float32),
    )(i0, i1, qT, kT, vT)


def _lepe_kernel(vp_ref, vc_ref, vn_ref, a0_ref, a1_ref, a2_ref, a3_ref,
                 wl_ref, bl_ref, woT_ref, bo_ref, o_ref, *, th, ntile):
    t = pl.program_id(1)
    acc = jnp.concatenate(
        [a0_ref[0, 0], a1_ref[0, 0], a2_ref[0, 0], a3_ref[0, 0]], axis=-1)
    cur = vc_ref[0]
    prev_last = vp_ref[0, th - 1:th] * jnp.where(t > 0, 1.0, 0.0)
    next_first = vn_ref[0, 0:1] * jnp.where(t < ntile - 1, 1.0, 0.0)
    rows = [
        jnp.concatenate([prev_last, cur[:th - 1]], axis=0),
        cur,
        jnp.concatenate([cur[1:], next_first], axis=0),
    ]
    for dy in range(3):
        vd = rows[dy]
        for dx in range(3):
            win = vd[:, dx:dx + HW_, :]
            lw = wl_ref[dy * 3 + dx][None, None, :]
            acc = acc + win * lw
    acc = acc + bl_ref[...][None]
    dn = (((2,), (0,)), ((), ()))
    out = jax.lax.dot_general(acc, woT_ref[...], dn, preferred_element_type=jnp.float32)
    o_ref[0] = out + bo_ref[...][None]


def _lepe_proj(vx, attn4, wl, bl2d, woT, bo2d):
    th = 28
    ntile = HW_ // th
    grid = (B_, ntile)
    vspec_p = pl.BlockSpec((1, th, HW_ + 2, DIM_),
                           lambda b, t: (b, jnp.maximum(t - 1, 0), 0, 0))
    vspec_c = pl.BlockSpec((1, th, HW_ + 2, DIM_), lambda b, t: (b, t, 0, 0))
    vspec_n = pl.BlockSpec((1, th, HW_ + 2, DIM_),
                           lambda b, t: (b, jnp.minimum(t + 1, ntile - 1), 0, 0))
    aspec = [pl.BlockSpec((1, 1, th, HW_, HD_),
                          lambda b, t, h=h: (b, h, t, 0, 0)) for h in range(NHEADS_)]
    return pl.pallas_call(
        functools.partial(_lepe_kernel, th=th, ntile=ntile),
        grid=grid,
        in_specs=[
            vspec_p, vspec_c, vspec_n,
            *aspec,
            pl.BlockSpec((9, DIM_), lambda b, t: (0, 0)),
            pl.BlockSpec((1, DIM_), lambda b, t: (0, 0)),
            pl.BlockSpec((DIM_, DIM_), lambda b, t: (0, 0)),
            pl.BlockSpec((1, DIM_), lambda b, t: (0, 0)),
        ],
        out_specs=pl.BlockSpec((1, th, HW_, DIM_), lambda b, t: (b, t, 0, 0)),
        out_shape=jax.ShapeDtypeStruct((B_, HW_, HW_, DIM_), jnp.float32),
    )(vx, vx, vx, attn4, attn4, attn4, attn4, wl, bl2d, woT, bo2d)


def kernel(x, Wqkv, bqkv, Wlepe, blepe, Wout, bout):
    qT5, kT5, vT5, vx, pooled = _qkv_proj(
        x, Wqkv.T, bqkv.reshape(1, 3 * DIM_))
    qk_flat = pooled.reshape(B_ * NREG_, 2 * DIM_)
    pooled3 = pooled.reshape(B_, NREG_, 2 * DIM_)

    i0, i1 = _route(qk_flat, pooled3)

    qT = qT5.reshape(B_ * NHEADS_, NREG_, HD_, RSS_)
    kT = kT5.reshape(B_ * NHEADS_, NREG_, HD_, RSS_)
    vT = vT5.reshape(B_ * NHEADS_, NREG_, HD_, RSS_)

    attn4 = _core_attn(qT, kT, vT, i0, i1)

    wl = Wlepe.reshape(DIM_, 9).T
    out = _lepe_proj(vx, attn4, wl, blepe.reshape(1, DIM_),
                     Wout.T, bout.reshape(1, DIM_))
    return out


# speedup vs baseline: 1.1148x; 1.0765x over previous
import functools

import jax
import jax.numpy as jnp
from jax.experimental import pallas as pl
from jax.experimental.pallas import tpu as pltpu

DIM_ = 96
NHEADS_ = 4
NWIN_ = 28
HD_ = DIM_ // NHEADS_
NREG_ = NWIN_ * NWIN_
RSS_ = 64
TOPK_ = 2
K2_ = TOPK_ * RSS_ // 8
NEG_ = -1e30

B_ = 2
HW_ = 224


def _qkv_kernel(x_ref, w_ref, b_ref, qT_ref, kT_ref, vT_ref, vx_ref,
                pool_ref, *, regs):
    x = x_ref[0].reshape(8 * HW_, DIM_)
    acc = jnp.dot(x, w_ref[...], preferred_element_type=jnp.float32)
    acc = acc + b_ref[...]
    acc4 = acc.reshape(8, regs, 8, 3 * DIM_)
    pool_ref[0] = jnp.mean(acc4, axis=(0, 2))[:, :2 * DIM_]
    accr = acc4.transpose(1, 0, 2, 3).reshape(regs, RSS_, 3 * DIM_)
    accT = jnp.swapaxes(accr, 1, 2)
    for h in range(NHEADS_):
        qT_ref[0, h] = accT[:, h * HD_:(h + 1) * HD_, :]
        kT_ref[0, h] = accT[:, DIM_ + h * HD_:DIM_ + (h + 1) * HD_, :]
        vT_ref[0, h] = accT[:, 2 * DIM_ + h * HD_:2 * DIM_ + (h + 1) * HD_, :]
    v3 = acc[:, 2 * DIM_:].reshape(8, HW_, DIM_)
    z = jnp.zeros((8, 1, DIM_), jnp.float32)
    vx_ref[0] = jnp.concatenate([z, v3, z], axis=1)


def _qkv_proj(x, wT, b2d):
    regs = NWIN_
    nchunk = NREG_ // regs
    grid = B_ * nchunk
    tblspec = pl.BlockSpec((1, NHEADS_, regs, HD_, RSS_),
                           lambda t: (t // nchunk, 0, t % nchunk, 0, 0))
    tblshape = jax.ShapeDtypeStruct((B_, NHEADS_, NREG_, HD_, RSS_), jnp.float32)
    return pl.pallas_call(
        functools.partial(_qkv_kernel, regs=regs),
        grid=(grid,),
        in_specs=[
            pl.BlockSpec((1, 8, HW_, DIM_),
                         lambda t: (t // nchunk, t % nchunk, 0, 0)),
            pl.BlockSpec((DIM_, 3 * DIM_), lambda t: (0, 0)),
            pl.BlockSpec((1, 3 * DIM_), lambda t: (0, 0)),
        ],
        out_specs=[
            tblspec, tblspec, tblspec,
            pl.BlockSpec((1, 8, HW_ + 2, DIM_),
                         lambda t: (t // nchunk, t % nchunk, 0, 0)),
            pl.BlockSpec((1, regs, 2 * DIM_), lambda t: (t, 0, 0)),
        ],
        out_shape=[
            tblshape, tblshape, tblshape,
            jax.ShapeDtypeStruct((B_, HW_, HW_ + 2, DIM_), jnp.float32),
            jax.ShapeDtypeStruct((grid, regs, 2 * DIM_), jnp.float32),
        ],
    )(x, wT, b2d)


def _route_kernel(qk_ref, kr_ref, i0_ref, i1_ref):
    q = qk_ref[:, :DIM_]
    kr = kr_ref[0, :, DIM_:]
    a = jax.lax.dot_general(q, kr, (((1,), (1,)), ((), ())),
                            preferred_element_type=jnp.float32)
    iota = jax.lax.broadcasted_iota(jnp.int32, a.shape, 1)
    m0 = jnp.max(a, axis=1, keepdims=True)
    i0 = jnp.min(jnp.where(a >= m0, iota, NREG_), axis=1, keepdims=True)
    a2 = jnp.where(iota == i0, NEG_, a)
    m1 = jnp.max(a2, axis=1, keepdims=True)
    i1 = jnp.min(jnp.where(a2 >= m1, iota, NREG_), axis=1, keepdims=True)
    i0_ref[0] = i0
    i1_ref[0] = i1


def _route(qk_flat, pooled3):
    rows = 112
    nchunk = NREG_ // rows
    grid = B_ * nchunk
    i0, i1 = pl.pallas_call(
        _route_kernel,
        grid=(grid,),
        in_specs=[
            pl.BlockSpec((rows, 2 * DIM_), lambda t: (t, 0)),
            pl.BlockSpec((1, NREG_, 2 * DIM_), lambda t: (t // nchunk, 0, 0)),
        ],
        out_specs=[
            pl.BlockSpec((1, rows, 1), lambda t: (t, 0, 0)),
            pl.BlockSpec((1, rows, 1), lambda t: (t, 0, 0)),
        ],
        out_shape=[
            jax.ShapeDtypeStruct((grid, rows, 1), jnp.int32),
            jax.ShapeDtypeStruct((grid, rows, 1), jnp.int32),
        ],
    )(qk_flat, pooled3)
    return i0.reshape(B_, NREG_), i1.reshape(B_, NREG_)


def _attn_kernel(i0_ref, i1_ref, q_ref, k_ref, v_ref, o_ref, *, scale, nrows):
    bh = pl.program_id(0)
    rh = pl.program_id(1)
    b = bh // NHEADS_
    nreg = nrows * NWIN_
    npair = nreg // 2

    dn = (((0,), (0,)), ((), ()))
    dn2 = (((0,), (1,)), ((), ()))
    gsize = npair
    outs = [None] * nreg
    for g in range(npair // gsize):
        idx = []
        origs = []
        for p in range(g * gsize, (g + 1) * gsize):
            rA = rh * nreg + 2 * p
            rB = rA + 1
            qA = q_ref[0, 2 * p] * scale
            qB = q_ref[0, 2 * p + 1] * scale
            iA0 = i0_ref[b, rA]
            iA1 = i1_ref[b, rA]
            iB0 = i0_ref[b, rB]
            iB1 = i1_ref[b, rB]
            idx.append((iA0, iA1, iB0, iB1))
            tA = jnp.concatenate(
                [jax.lax.dot_general(k_ref[0, iA0], qA, dn, preferred_element_type=jnp.float32),
                 jax.lax.dot_general(k_ref[0, iA1], qA, dn, preferred_element_type=jnp.float32)],
                axis=0)
            tB = jnp.concatenate(
                [jax.lax.dot_general(k_ref[0, iB0], qB, dn, preferred_element_type=jnp.float32),
                 jax.lax.dot_general(k_ref[0, iB1], qB, dn, preferred_element_type=jnp.float32)],
                axis=0)
            origs.append(jnp.concatenate([tA, tB], axis=1))

        ts = list(origs)
        ms = [None] * gsize
        for it in range(K2_ - 1):
            for j in range(gsize):
                cm = jnp.max(ts[j], axis=0, keepdims=True)
                if it == 0:
                    ms[j] = cm
                ts[j] = jnp.where(ts[j] >= cm, NEG_, ts[j])

        for j in range(gsize):
            p = g * gsize + j
            orig = origs[j]
            iA0, iA1, iB0, iB1 = idx[j]
            thresh = jnp.max(ts[j], axis=0, keepdims=True)
            e = jnp.where(orig >= thresh, jnp.exp(orig - ms[j]), 0.0)
            s = jnp.sum(e, axis=0, keepdims=True)
            pn = e / s
            vA = jnp.concatenate([v_ref[0, iA0], v_ref[0, iA1]], axis=1)
            vB = jnp.concatenate([v_ref[0, iB0], v_ref[0, iB1]], axis=1)
            oA = jax.lax.dot_general(pn[:, :64], vA, dn2, preferred_element_type=jnp.float32)
            oB = jax.lax.dot_general(pn[:, 64:], vB, dn2, preferred_element_type=jnp.float32)
            outs[2 * p] = oA
            outs[2 * p + 1] = oB
    o5 = (jnp.stack(outs).reshape(nrows, NWIN_, 8, 8, HD_)
          .transpose(0, 2, 1, 3, 4))
    o_ref[0, 0] = o5.reshape(nrows * 8, HW_, HD_)


def _core_attn(qT, kT, vT, i0, i1):
    scale = DIM_ ** (-0.5)
    nrows = 2
    grid_spec = pltpu.PrefetchScalarGridSpec(
        num_scalar_prefetch=2,
        grid=(B_ * NHEADS_, NWIN_ // nrows),
        in_specs=[
            pl.BlockSpec((1, nrows * NWIN_, HD_, RSS_),
                         lambda bh, c, i0, i1: (bh, c, 0, 0)),
            pl.BlockSpec((1, NREG_, HD_, RSS_), lambda bh, c, i0, i1: (bh, 0, 0, 0)),
            pl.BlockSpec((1, NREG_, HD_, RSS_), lambda bh, c, i0, i1: (bh, 0, 0, 0)),
        ],
        out_specs=pl.BlockSpec(
            (1, 1, nrows * 8, HW_, HD_),
            lambda bh, c, i0, i1: (bh // NHEADS_, bh % NHEADS_, c, 0, 0)),
    )
    return pl.pallas_call(
        functools.partial(_attn_kernel, scale=scale, nrows=nrows),
        grid_spec=grid_spec,
        out_shape=jax.ShapeDtypeStruct((B_, NHEADS_, HW_, HW_, HD_), jnp.float32),
    )(i0, i1, qT, kT, vT)


def _lepe_kernel(vp_ref, vc_ref, vn_ref, a0_ref, a1_ref, a2_ref, a3_ref,
                 wl_ref, bl_ref, woT_ref, bo_ref, o_ref, *, th, ntile):
    t = pl.program_id(1)
    acc = jnp.concatenate(
        [a0_ref[0, 0], a1_ref[0, 0], a2_ref[0, 0], a3_ref[0, 0]], axis=-1)
    cur = vc_ref[0]
    prev_last = vp_ref[0, th - 1:th] * jnp.where(t > 0, 1.0, 0.0)
    next_first = vn_ref[0, 0:1] * jnp.where(t < ntile - 1, 1.0, 0.0)
    rows = [
        jnp.concatenate([prev_last, cur[:th - 1]], axis=0),
        cur,
        jnp.concatenate([cur[1:], next_first], axis=0),
    ]
    for dy in range(3):
        vd = rows[dy]
        for dx in range(3):
            win = vd[:, dx:dx + HW_, :]
            lw = wl_ref[dy * 3 + dx][None, None, :]
            acc = acc + win * lw
    acc = acc + bl_ref[...][None]
    dn = (((2,), (0,)), ((), ()))
    out = jax.lax.dot_general(acc, woT_ref[...], dn, preferred_element_type=jnp.float32)
    o_ref[0] = out + bo_ref[...][None]


def _lepe_proj(vx, attn4, wl, bl2d, woT, bo2d):
    th = 28
    ntile = HW_ // th
    grid = (B_, ntile)
    vspec_p = pl.BlockSpec((1, th, HW_ + 2, DIM_),
                           lambda b, t: (b, jnp.maximum(t - 1, 0), 0, 0))
    vspec_c = pl.BlockSpec((1, th, HW_ + 2, DIM_), lambda b, t: (b, t, 0, 0))
    vspec_n = pl.BlockSpec((1, th, HW_ + 2, DIM_),
                           lambda b, t: (b, jnp.minimum(t + 1, ntile - 1), 0, 0))
    aspec = [pl.BlockSpec((1, 1, th, HW_, HD_),
                          lambda b, t, h=h: (b, h, t, 0, 0)) for h in range(NHEADS_)]
    return pl.pallas_call(
        functools.partial(_lepe_kernel, th=th, ntile=ntile),
        grid=grid,
        in_specs=[
            vspec_p, vspec_c, vspec_n,
            *aspec,
            pl.BlockSpec((9, DIM_), lambda b, t: (0, 0)),
            pl.BlockSpec((1, DIM_), lambda b, t: (0, 0)),
            pl.BlockSpec((DIM_, DIM_), lambda b, t: (0, 0)),
            pl.BlockSpec((1, DIM_), lambda b, t: (0, 0)),
        ],
        out_specs=pl.BlockSpec((1, th, HW_, DIM_), lambda b, t: (b, t, 0, 0)),
        out_shape=jax.ShapeDtypeStruct((B_, HW_, HW_, DIM_), jnp.float32),
    )(vx, vx, vx, attn4, attn4, attn4, attn4, wl, bl2d, woT, bo2d)


def kernel(x, Wqkv, bqkv, Wlepe, blepe, Wout, bout):
    qT5, kT5, vT5, vx, pooled = _qkv_proj(
        x, Wqkv.T, bqkv.reshape(1, 3 * DIM_))
    qk_flat = pooled.reshape(B_ * NREG_, 2 * DIM_)
    pooled3 = pooled.reshape(B_, NREG_, 2 * DIM_)

    i0, i1 = _route(qk_flat, pooled3)

    qT = qT5.reshape(B_ * NHEADS_, NREG_, HD_, RSS_)
    kT = kT5.reshape(B_ * NHEADS_, NREG_, HD_, RSS_)
    vT = vT5.reshape(B_ * NHEADS_, NREG_, HD_, RSS_)

    attn4 = _core_attn(qT, kT, vT, i0, i1)

    wl = Wlepe.reshape(DIM_, 9).T
    out = _lepe_proj(vx, attn4, wl, blepe.reshape(1, DIM_),
                     Wout.T, bout.reshape(1, DIM_))
    return out


# speedup vs baseline: 1.1603x; 1.0408x over previous
import functools

import jax
import jax.numpy as jnp
from jax.experimental import pallas as pl
from jax.experimental.pallas import tpu as pltpu

DIM_ = 96
NHEADS_ = 4
NWIN_ = 28
HD_ = DIM_ // NHEADS_
NREG_ = NWIN_ * NWIN_
RSS_ = 64
TOPK_ = 2
K2_ = TOPK_ * RSS_ // 8
NEG_ = -1e30

B_ = 2
HW_ = 224


def _qkv_kernel(x_ref, w_ref, b_ref, qT_ref, kT_ref, vT_ref, vx_ref,
                pool_ref, *, regs):
    x = x_ref[0].reshape(8 * HW_, DIM_)
    acc = jnp.dot(x, w_ref[...], preferred_element_type=jnp.float32)
    acc = acc + b_ref[...]
    acc4 = acc.reshape(8, regs, 8, 3 * DIM_)
    pool_ref[0] = jnp.mean(acc4, axis=(0, 2))[:, :2 * DIM_]
    accr = acc4.transpose(1, 0, 2, 3).reshape(regs, RSS_, 3 * DIM_)
    accT = jnp.swapaxes(accr, 1, 2)
    for h in range(NHEADS_):
        qT_ref[0, h] = accT[:, h * HD_:(h + 1) * HD_, :]
        kT_ref[0, h] = accT[:, DIM_ + h * HD_:DIM_ + (h + 1) * HD_, :]
        vT_ref[0, h] = accT[:, 2 * DIM_ + h * HD_:2 * DIM_ + (h + 1) * HD_, :]
    v3 = acc[:, 2 * DIM_:].reshape(8, HW_, DIM_)
    z = jnp.zeros((8, 1, DIM_), jnp.float32)
    vx_ref[0] = jnp.concatenate([z, v3, z], axis=1)


def _qkv_proj(x, wT, b2d):
    regs = NWIN_
    nchunk = NREG_ // regs
    grid = B_ * nchunk
    tblspec = pl.BlockSpec((1, NHEADS_, regs, HD_, RSS_),
                           lambda t: (t // nchunk, 0, t % nchunk, 0, 0))
    tblshape = jax.ShapeDtypeStruct((B_, NHEADS_, NREG_, HD_, RSS_), jnp.float32)
    return pl.pallas_call(
        functools.partial(_qkv_kernel, regs=regs),
        grid=(grid,),
        in_specs=[
            pl.BlockSpec((1, 8, HW_, DIM_),
                         lambda t: (t // nchunk, t % nchunk, 0, 0)),
            pl.BlockSpec((DIM_, 3 * DIM_), lambda t: (0, 0)),
            pl.BlockSpec((1, 3 * DIM_), lambda t: (0, 0)),
        ],
        out_specs=[
            tblspec, tblspec, tblspec,
            pl.BlockSpec((1, 8, HW_ + 2, DIM_),
                         lambda t: (t // nchunk, t % nchunk, 0, 0)),
            pl.BlockSpec((1, regs, 2 * DIM_), lambda t: (t, 0, 0)),
        ],
        out_shape=[
            tblshape, tblshape, tblshape,
            jax.ShapeDtypeStruct((B_, HW_, HW_ + 2, DIM_), jnp.float32),
            jax.ShapeDtypeStruct((grid, regs, 2 * DIM_), jnp.float32),
        ],
    )(x, wT, b2d)


def _route_kernel(qk_ref, kr_ref, i0_ref, i1_ref):
    q = qk_ref[:, :DIM_]
    kr = kr_ref[0, :, DIM_:]
    a = jax.lax.dot_general(q, kr, (((1,), (1,)), ((), ())),
                            preferred_element_type=jnp.float32)
    iota = jax.lax.broadcasted_iota(jnp.int32, a.shape, 1)
    m0 = jnp.max(a, axis=1, keepdims=True)
    i0 = jnp.min(jnp.where(a >= m0, iota, NREG_), axis=1, keepdims=True)
    a2 = jnp.where(iota == i0, NEG_, a)
    m1 = jnp.max(a2, axis=1, keepdims=True)
    i1 = jnp.min(jnp.where(a2 >= m1, iota, NREG_), axis=1, keepdims=True)
    i0_ref[0] = i0
    i1_ref[0] = i1


def _route(qk_flat, pooled3):
    rows = 112
    nchunk = NREG_ // rows
    grid = B_ * nchunk
    i0, i1 = pl.pallas_call(
        _route_kernel,
        grid=(grid,),
        in_specs=[
            pl.BlockSpec((rows, 2 * DIM_), lambda t: (t, 0)),
            pl.BlockSpec((1, NREG_, 2 * DIM_), lambda t: (t // nchunk, 0, 0)),
        ],
        out_specs=[
            pl.BlockSpec((1, rows, 1), lambda t: (t, 0, 0)),
            pl.BlockSpec((1, rows, 1), lambda t: (t, 0, 0)),
        ],
        out_shape=[
            jax.ShapeDtypeStruct((grid, rows, 1), jnp.int32),
            jax.ShapeDtypeStruct((grid, rows, 1), jnp.int32),
        ],
    )(qk_flat, pooled3)
    return i0.reshape(B_, NREG_), i1.reshape(B_, NREG_)


def _attn_kernel(i0_ref, i1_ref, q_ref, k_ref, v_ref, o_ref, *, scale, nrows):
    bh = pl.program_id(0)
    rh = pl.program_id(1)
    b = bh // NHEADS_
    nreg = nrows * NWIN_
    npair = nreg // 2

    dn = (((0,), (0,)), ((), ()))
    dn2 = (((0,), (1,)), ((), ()))
    gsize = npair
    outs = [None] * nreg
    for g in range(npair // gsize):
        idx = []
        origs = []
        for p in range(g * gsize, (g + 1) * gsize):
            rA = rh * nreg + 2 * p
            rB = rA + 1
            qA = q_ref[0, 2 * p] * scale
            qB = q_ref[0, 2 * p + 1] * scale
            iA0 = i0_ref[b, rA]
            iA1 = i1_ref[b, rA]
            iB0 = i0_ref[b, rB]
            iB1 = i1_ref[b, rB]
            idx.append((iA0, iA1, iB0, iB1))
            tA = jnp.concatenate(
                [jax.lax.dot_general(k_ref[0, iA0], qA, dn, preferred_element_type=jnp.float32),
                 jax.lax.dot_general(k_ref[0, iA1], qA, dn, preferred_element_type=jnp.float32)],
                axis=0)
            tB = jnp.concatenate(
                [jax.lax.dot_general(k_ref[0, iB0], qB, dn, preferred_element_type=jnp.float32),
                 jax.lax.dot_general(k_ref[0, iB1], qB, dn, preferred_element_type=jnp.float32)],
                axis=0)
            origs.append(jnp.concatenate([tA, tB], axis=1))

        ts = list(origs)
        ms = [None] * gsize
        for it in range(K2_ - 1):
            for j in range(gsize):
                cm = jnp.max(ts[j], axis=0, keepdims=True)
                if it == 0:
                    ms[j] = cm
                ts[j] = jnp.where(ts[j] >= cm, NEG_, ts[j])

        for j in range(gsize):
            p = g * gsize + j
            orig = origs[j]
            iA0, iA1, iB0, iB1 = idx[j]
            thresh = jnp.max(ts[j], axis=0, keepdims=True)
            e = jnp.where(orig >= thresh, jnp.exp(orig - ms[j]), 0.0)
            s = jnp.sum(e, axis=0, keepdims=True)
            pn = e / s
            vA = jnp.concatenate([v_ref[0, iA0], v_ref[0, iA1]], axis=1)
            vB = jnp.concatenate([v_ref[0, iB0], v_ref[0, iB1]], axis=1)
            oA = jax.lax.dot_general(pn[:, :64], vA, dn2, preferred_element_type=jnp.float32)
            oB = jax.lax.dot_general(pn[:, 64:], vB, dn2, preferred_element_type=jnp.float32)
            outs[2 * p] = oA
            outs[2 * p + 1] = oB
    o5 = (jnp.stack(outs).reshape(nrows, NWIN_, 8, 8, HD_)
          .transpose(0, 2, 1, 3, 4))
    o_ref[0, 0] = o5.reshape(nrows * 8, HW_, HD_)


def _core_attn(qT, kT, vT, i0, i1):
    scale = DIM_ ** (-0.5)
    nrows = 4
    grid_spec = pltpu.PrefetchScalarGridSpec(
        num_scalar_prefetch=2,
        grid=(B_ * NHEADS_, NWIN_ // nrows),
        in_specs=[
            pl.BlockSpec((1, nrows * NWIN_, HD_, RSS_),
                         lambda bh, c, i0, i1: (bh, c, 0, 0)),
            pl.BlockSpec((1, NREG_, HD_, RSS_), lambda bh, c, i0, i1: (bh, 0, 0, 0)),
            pl.BlockSpec((1, NREG_, HD_, RSS_), lambda bh, c, i0, i1: (bh, 0, 0, 0)),
        ],
        out_specs=pl.BlockSpec(
            (1, 1, nrows * 8, HW_, HD_),
            lambda bh, c, i0, i1: (bh // NHEADS_, bh % NHEADS_, c, 0, 0)),
    )
    return pl.pallas_call(
        functools.partial(_attn_kernel, scale=scale, nrows=nrows),
        grid_spec=grid_spec,
        out_shape=jax.ShapeDtypeStruct((B_, NHEADS_, HW_, HW_, HD_), jnp.float32),
    )(i0, i1, qT, kT, vT)


def _lepe_kernel(vp_ref, vc_ref, vn_ref, a0_ref, a1_ref, a2_ref, a3_ref,
                 wl_ref, bl_ref, woT_ref, bo_ref, o_ref, *, th, ntile):
    t = pl.program_id(1)
    acc = jnp.concatenate(
        [a0_ref[0, 0], a1_ref[0, 0], a2_ref[0, 0], a3_ref[0, 0]], axis=-1)
    cur = vc_ref[0]
    prev_last = vp_ref[0, th - 1:th] * jnp.where(t > 0, 1.0, 0.0)
    next_first = vn_ref[0, 0:1] * jnp.where(t < ntile - 1, 1.0, 0.0)
    rows = [
        jnp.concatenate([prev_last, cur[:th - 1]], axis=0),
        cur,
        jnp.concatenate([cur[1:], next_first], axis=0),
    ]
    for dy in range(3):
        vd = rows[dy]
        for dx in range(3):
            win = vd[:, dx:dx + HW_, :]
            lw = wl_ref[dy * 3 + dx][None, None, :]
            acc = acc + win * lw
    acc = acc + bl_ref[...][None]
    dn = (((2,), (0,)), ((), ()))
    out = jax.lax.dot_general(acc, woT_ref[...], dn, preferred_element_type=jnp.float32)
    o_ref[0] = out + bo_ref[...][None]


def _lepe_proj(vx, attn4, wl, bl2d, woT, bo2d):
    th = 28
    ntile = HW_ // th
    grid = (B_, ntile)
    vspec_p = pl.BlockSpec((1, th, HW_ + 2, DIM_),
                           lambda b, t: (b, jnp.maximum(t - 1, 0), 0, 0))
    vspec_c = pl.BlockSpec((1, th, HW_ + 2, DIM_), lambda b, t: (b, t, 0, 0))
    vspec_n = pl.BlockSpec((1, th, HW_ + 2, DIM_),
                           lambda b, t: (b, jnp.minimum(t + 1, ntile - 1), 0, 0))
    aspec = [pl.BlockSpec((1, 1, th, HW_, HD_),
                          lambda b, t, h=h: (b, h, t, 0, 0)) for h in range(NHEADS_)]
    return pl.pallas_call(
        functools.partial(_lepe_kernel, th=th, ntile=ntile),
        grid=grid,
        in_specs=[
            vspec_p, vspec_c, vspec_n,
            *aspec,
            pl.BlockSpec((9, DIM_), lambda b, t: (0, 0)),
            pl.BlockSpec((1, DIM_), lambda b, t: (0, 0)),
            pl.BlockSpec((DIM_, DIM_), lambda b, t: (0, 0)),
            pl.BlockSpec((1, DIM_), lambda b, t: (0, 0)),
        ],
        out_specs=pl.BlockSpec((1, th, HW_, DIM_), lambda b, t: (b, t, 0, 0)),
        out_shape=jax.ShapeDtypeStruct((B_, HW_, HW_, DIM_), jnp.float32),
    )(vx, vx, vx, attn4, attn4, attn4, attn4, wl, bl2d, woT, bo2d)


def kernel(x, Wqkv, bqkv, Wlepe, blepe, Wout, bout):
    qT5, kT5, vT5, vx, pooled = _qkv_proj(
        x, Wqkv.T, bqkv.reshape(1, 3 * DIM_))
    qk_flat = pooled.reshape(B_ * NREG_, 2 * DIM_)
    pooled3 = pooled.reshape(B_, NREG_, 2 * DIM_)

    i0, i1 = _route(qk_flat, pooled3)

    qT = qT5.reshape(B_ * NHEADS_, NREG_, HD_, RSS_)
    kT = kT5.reshape(B_ * NHEADS_, NREG_, HD_, RSS_)
    vT = vT5.reshape(B_ * NHEADS_, NREG_, HD_, RSS_)

    attn4 = _core_attn(qT, kT, vT, i0, i1)

    wl = Wlepe.reshape(DIM_, 9).T
    out = _lepe_proj(vx, attn4, wl, blepe.reshape(1, DIM_),
                     Wout.T, bout.reshape(1, DIM_))
    return out
